# baseline (speedup 1.0000x reference)
import functools

import jax
import jax.numpy as jnp
from jax import lax
from jax.experimental import pallas as pl
from jax.experimental.pallas import tpu as pltpu
from jax.experimental.pallas import tpu_sc as plsc

B, H, W, C = 8, 224, 224, 3
PS = 4
VOCAB = 512
D = 256
PD = PS * PS * C
Hp = H // PS
Wp = W // PS
N = Hp * Wp
R = B * N

BLK = 1568
G = R // BLK
LC = PS * C

_INV_SQRT2 = 0.7071067811865476


def _gelu(x):
    return x * 0.5 * (1.0 + jax.lax.erf(x * _INV_SQRT2))


def _table_body(cb, dw1, db1, dw2, db2, dw3, db3, ptable_out, cn_out):
    codebook = cb[...]
    cn_out[...] = jnp.sum(codebook * codebook, axis=-1)[None, :]
    x = _gelu(jnp.dot(codebook, dw1[...], preferred_element_type=jnp.float32) + db1[...])
    x = _gelu(jnp.dot(x, dw2[...], preferred_element_type=jnp.float32) + db2[...])
    ptable_out[...] = jnp.dot(x, dw3[...], preferred_element_type=jnp.float32) + db3[...]


def _main_body(praw_ref, ew1, eb1, ew2, eb2, ew3, eb3, cb, cn_ref, pt_ref,
               p_out, tok_out, loss_out):
    i = pl.program_id(0)
    t = praw_ref[...] / 255.0 * 2.0 - 1.0
    z = _gelu(jnp.dot(t, ew1[...], preferred_element_type=jnp.float32) + eb1[...])
    z = _gelu(jnp.dot(z, ew2[...], preferred_element_type=jnp.float32) + eb2[...])
    z_e = jnp.dot(z, ew3[...], preferred_element_type=jnp.float32) + eb3[...]

    score = jnp.dot(z_e, cb[...].T, preferred_element_type=jnp.float32)
    g = cn_ref[...] - 2.0 * score

    m = jnp.min(g, axis=-1, keepdims=True)
    iota = jax.lax.broadcasted_iota(jnp.int32, g.shape, 1)
    tok = jnp.min(jnp.where(g == m, iota, VOCAB), axis=-1)
    tok_out[0, 0, :] = tok

    onehot = (iota == tok[:, None]).astype(jnp.float32)
    p = jnp.dot(onehot, pt_ref[...], preferred_element_type=jnp.float32)
    p_out[...] = p

    zn = jnp.sum(z_e * z_e, axis=-1, keepdims=True)
    vq_sum = jnp.sum(zn + m)
    rec_sum = jnp.sum((p - t) ** 2)

    @pl.when(i == 0)
    def _init():
        loss_out[...] = jnp.zeros_like(loss_out)

    upd = jnp.concatenate([rec_sum.reshape(1, 1), vq_sum.reshape(1, 1)], axis=1)
    loss_out[...] += upd


ROWS = B * H
RW = W * C
NV = RW // 16
NW = 32
RPW = ROWS // NW
TPW = RPW // PS * Wp


def _sc_unpatch_body(pt_hbm, tok_hbm, wp_hbm, col_hbm, out_hbm,
                     pt_v, tok_v, wp_v, col_v, buf_v):
    wid = lax.axis_index("s") * 2 + lax.axis_index("c")
    pltpu.sync_copy(pt_hbm, pt_v)
    pltpu.sync_copy(wp_hbm, wp_v)
    pltpu.sync_copy(col_hbm, col_v)
    pltpu.sync_copy(tok_hbm.at[pl.ds(wid * TPW, TPW)], tok_v)

    def body(r, carry):
        hp_l = r // PS
        p1 = r % PS
        for v in range(NV):
            wp = wp_v[v]
            rvec = plsc.load_gather(tok_v, [hp_l * Wp + wp])
            cvec = col_v[v] + p1 * LC
            vals = plsc.load_gather(pt_v, [rvec, cvec])
            buf_v[r, v * 16:(v + 1) * 16] = vals
        return carry

    lax.fori_loop(0, RPW, body, 0)
    pltpu.sync_copy(buf_v, out_hbm.at[pl.ds(wid * RPW, RPW)])


@functools.partial(
    pl.kernel,
    mesh=plsc.VectorSubcoreMesh(core_axis_name="c", subcore_axis_name="s"),
    out_type=jax.ShapeDtypeStruct((ROWS, RW), jnp.float32),
    compiler_params=pltpu.CompilerParams(needs_layout_passes=False),
    scratch_types=[
        pltpu.VMEM((VOCAB, PD), jnp.float32),
        pltpu.VMEM((TPW,), jnp.int32),
        pltpu.VMEM((NV, 16), jnp.int32),
        pltpu.VMEM((NV, 16), jnp.int32),
        pltpu.VMEM((RPW, RW), jnp.float32),
    ],
)
def _sc_unpatch(pt_hbm, tok_hbm, wp_hbm, col_hbm, out_hbm,
                pt_v, tok_v, wp_v, col_v, buf_v):
    _sc_unpatch_body(pt_hbm, tok_hbm, wp_hbm, col_hbm, out_hbm,
                     pt_v, tok_v, wp_v, col_v, buf_v)


def kernel(frames, enc_w1, enc_b1, enc_w2, enc_b2, enc_w3, enc_b3, codebook,
           dec_w1, dec_b1, dec_w2, dec_b2, dec_w3, dec_b3):
    praw = frames.astype(jnp.float32).reshape(B, Hp, PS, Wp, PS, C)
    praw = praw.transpose(0, 1, 3, 2, 4, 5).reshape(R, PD)

    full = lambda shape: pl.BlockSpec(shape, lambda i: (0,) * len(shape))

    ptable, cn = pl.pallas_call(
        _table_body,
        grid=(1,),
        in_specs=[full((VOCAB, D)), full((D, D)), full((1, D)), full((D, D)),
                  full((1, D)), full((D, PD)), full((1, PD))],
        out_specs=(full((VOCAB, PD)), full((1, VOCAB))),
        out_shape=(jax.ShapeDtypeStruct((VOCAB, PD), jnp.float32),
                   jax.ShapeDtypeStruct((1, VOCAB), jnp.float32)),
    )(codebook, dec_w1, dec_b1.reshape(1, D), dec_w2, dec_b2.reshape(1, D),
      dec_w3, dec_b3.reshape(1, PD))

    bspecs = [
        pl.BlockSpec((BLK, PD), lambda i: (i, 0)),
        full((PD, D)), full((1, D)),
        full((D, D)), full((1, D)),
        full((D, D)), full((1, D)),
        full((VOCAB, D)),
        full((1, VOCAB)),
        full((VOCAB, PD)),
    ]
    out_shapes = (
        jax.ShapeDtypeStruct((R, PD), jnp.float32),
        jax.ShapeDtypeStruct((G, 1, BLK), jnp.int32),
        jax.ShapeDtypeStruct((1, 2), jnp.float32),
    )
    out_specs = (
        pl.BlockSpec((BLK, PD), lambda i: (i, 0)),
        pl.BlockSpec((1, 1, BLK), lambda i: (i, 0, 0)),
        pl.BlockSpec((1, 2), lambda i: (0, 0)),
    )
    p_full, tok3, sums = pl.pallas_call(
        _main_body,
        grid=(G,),
        in_specs=bspecs,
        out_specs=out_specs,
        out_shape=out_shapes,
    )(praw, enc_w1, enc_b1.reshape(1, D), enc_w2, enc_b2.reshape(1, D),
      enc_w3, enc_b3.reshape(1, D), codebook, cn, ptable)

    tokens = tok3.reshape(B, N)

    lane = jnp.arange(RW, dtype=jnp.int32)
    wp_pat = (lane // LC).reshape(NV, 16)
    col_pat = (lane % LC).reshape(NV, 16)
    rec2d = _sc_unpatch(ptable, tok3.reshape(R), wp_pat, col_pat)
    recon = rec2d.reshape(B, H, W, C)
    recon_loss = sums[0, 0] / (B * H * W * C)
    vq_loss = sums[0, 1] / (R * D)
    return (recon, tokens, recon_loss, vq_loss, vq_loss)


# speedup vs baseline: 1.1231x; 1.1231x over previous
import functools

import jax
import jax.numpy as jnp
from jax import lax
from jax.experimental import pallas as pl
from jax.experimental.pallas import tpu as pltpu
from jax.experimental.pallas import tpu_sc as plsc

B, H, W, C = 8, 224, 224, 3
PS = 4
VOCAB = 512
D = 256
PD = PS * PS * C
Hp = H // PS
Wp = W // PS
N = Hp * Wp
R = B * N

BLK = 1568
G = R // BLK
LC = PS * C

_INV_SQRT2 = 0.7071067811865476


def _gelu(x):
    return x * 0.5 * (1.0 + jax.lax.erf(x * _INV_SQRT2))


def _table_body(cb, dw1, db1, dw2, db2, dw3, db3, ptable_out, cn_out):
    codebook = cb[...]
    cn_out[...] = jnp.sum(codebook * codebook, axis=-1)[None, :]
    x = _gelu(jnp.dot(codebook, dw1[...], preferred_element_type=jnp.float32) + db1[...])
    x = _gelu(jnp.dot(x, dw2[...], preferred_element_type=jnp.float32) + db2[...])
    ptable_out[...] = jnp.dot(x, dw3[...], preferred_element_type=jnp.float32) + db3[...]


def _main_body(praw_ref, ew1, eb1, ew2, eb2, ew3, eb3, cb, cn_ref, pt_ref,
               p_out, tok_out, loss_out):
    i = pl.program_id(0)
    t = praw_ref[...] / 255.0 * 2.0 - 1.0
    z = _gelu(jnp.dot(t, ew1[...], preferred_element_type=jnp.float32) + eb1[...])
    z = _gelu(jnp.dot(z, ew2[...], preferred_element_type=jnp.float32) + eb2[...])
    z_e = jnp.dot(z, ew3[...], preferred_element_type=jnp.float32) + eb3[...]

    score = jnp.dot(z_e, cb[...].T, preferred_element_type=jnp.float32)
    g = cn_ref[...] - 2.0 * score

    m = jnp.min(g, axis=-1, keepdims=True)
    iota = jax.lax.broadcasted_iota(jnp.int32, g.shape, 1)
    tok = jnp.min(jnp.where(g == m, iota, VOCAB), axis=-1)
    tok_out[0, 0, :] = tok

    onehot = (iota == tok[:, None]).astype(jnp.float32)
    p = jnp.dot(onehot, pt_ref[...], preferred_element_type=jnp.float32)
    p_out[...] = p

    zn = jnp.sum(z_e * z_e, axis=-1, keepdims=True)
    vq_sum = jnp.sum(zn + m)
    rec_sum = jnp.sum((p - t) ** 2)

    @pl.when(i == 0)
    def _init():
        loss_out[...] = jnp.zeros_like(loss_out)

    upd = jnp.concatenate([rec_sum.reshape(1, 1), vq_sum.reshape(1, 1)], axis=1)
    loss_out[...] += upd


NVP = PD // 16
QP = R // 4
PPW = 4 * Wp
NWQ = QP // PPW


def _sc_patch_body(fr_hbm, rp_hbm, cp_hbm, out_hbm, src_v, rp_v, cp_v, buf_v):
    wid = lax.axis_index("s") * 2 + lax.axis_index("c")

    @pl.when(wid < NWQ)
    def _active():
        local_b = wid // 14
        hp0 = (wid % 14) * PS
        pltpu.sync_copy(rp_hbm, rp_v)
        pltpu.sync_copy(cp_hbm, cp_v)
        pltpu.sync_copy(fr_hbm.at[pl.ds(local_b * H + hp0 * PS, 4 * PS)], src_v)

        def body(r, carry):
            hp_l = r // Wp
            wp = r % Wp
            for v in range(NVP):
                srow = hp_l * PS + rp_v[v]
                scol = wp * LC + cp_v[v]
                vals = plsc.load_gather(src_v, [srow, scol])
                buf_v[r, v * 16:(v + 1) * 16] = vals
            return carry

        lax.fori_loop(0, PPW, body, 0)
        pltpu.sync_copy(buf_v, out_hbm.at[pl.ds(wid * PPW, PPW)])


@functools.partial(
    pl.kernel,
    mesh=plsc.VectorSubcoreMesh(core_axis_name="c", subcore_axis_name="s"),
    out_type=jax.ShapeDtypeStruct((QP, PD), jnp.float32),
    compiler_params=pltpu.CompilerParams(needs_layout_passes=False),
    scratch_types=[
        pltpu.VMEM((4 * PS, W * C), jnp.float32),
        pltpu.VMEM((NVP, 16), jnp.int32),
        pltpu.VMEM((NVP, 16), jnp.int32),
        pltpu.VMEM((PPW, PD), jnp.float32),
    ],
)
def _sc_patch(fr_hbm, rp_hbm, cp_hbm, out_hbm, src_v, rp_v, cp_v, buf_v):
    _sc_patch_body(fr_hbm, rp_hbm, cp_hbm, out_hbm, src_v, rp_v, cp_v, buf_v)


ROWS = B * H
RW = W * C
NV = RW // 16
NW = 32
RPW = ROWS // NW
TPW = RPW // PS * Wp


def _sc_unpatch_body(pt_hbm, tok_hbm, wp_hbm, col_hbm, out_hbm,
                     pt_v, tok_v, wp_v, col_v, buf_v):
    wid = lax.axis_index("s") * 2 + lax.axis_index("c")
    pltpu.sync_copy(pt_hbm, pt_v)
    pltpu.sync_copy(wp_hbm, wp_v)
    pltpu.sync_copy(col_hbm, col_v)
    pltpu.sync_copy(tok_hbm.at[pl.ds(wid * TPW, TPW)], tok_v)

    def body(r, carry):
        hp_l = r // PS
        p1 = r % PS
        for v in range(NV):
            wp = wp_v[v]
            rvec = plsc.load_gather(tok_v, [hp_l * Wp + wp])
            cvec = col_v[v] + p1 * LC
            vals = plsc.load_gather(pt_v, [rvec, cvec])
            buf_v[r, v * 16:(v + 1) * 16] = vals
        return carry

    lax.fori_loop(0, RPW, body, 0)
    pltpu.sync_copy(buf_v, out_hbm.at[pl.ds(wid * RPW, RPW)])


@functools.partial(
    pl.kernel,
    mesh=plsc.VectorSubcoreMesh(core_axis_name="c", subcore_axis_name="s"),
    out_type=jax.ShapeDtypeStruct((ROWS, RW), jnp.float32),
    compiler_params=pltpu.CompilerParams(needs_layout_passes=False),
    scratch_types=[
        pltpu.VMEM((VOCAB, PD), jnp.float32),
        pltpu.VMEM((TPW,), jnp.int32),
        pltpu.VMEM((NV, 16), jnp.int32),
        pltpu.VMEM((NV, 16), jnp.int32),
        pltpu.VMEM((RPW, RW), jnp.float32),
    ],
)
def _sc_unpatch(pt_hbm, tok_hbm, wp_hbm, col_hbm, out_hbm,
                pt_v, tok_v, wp_v, col_v, buf_v):
    _sc_unpatch_body(pt_hbm, tok_hbm, wp_hbm, col_hbm, out_hbm,
                     pt_v, tok_v, wp_v, col_v, buf_v)


def kernel(frames, enc_w1, enc_b1, enc_w2, enc_b2, enc_w3, enc_b3, codebook,
           dec_w1, dec_b1, dec_w2, dec_b2, dec_w3, dec_b3):
    f2 = frames.astype(jnp.float32).reshape(B * H, W * C)
    pcol = jnp.arange(PD, dtype=jnp.int32)
    rp_pat = (pcol // LC).reshape(NVP, 16)
    cp_pat = (pcol % LC).reshape(NVP, 16)
    quarters = [
        _sc_patch(lax.slice_in_dim(f2, q * 2 * H, (q + 1) * 2 * H, axis=0),
                  rp_pat, cp_pat)
        for q in range(4)
    ]
    praw = jnp.concatenate(quarters, axis=0)

    full = lambda shape: pl.BlockSpec(shape, lambda i: (0,) * len(shape))

    ptable, cn = pl.pallas_call(
        _table_body,
        grid=(1,),
        in_specs=[full((VOCAB, D)), full((D, D)), full((1, D)), full((D, D)),
                  full((1, D)), full((D, PD)), full((1, PD))],
        out_specs=(full((VOCAB, PD)), full((1, VOCAB))),
        out_shape=(jax.ShapeDtypeStruct((VOCAB, PD), jnp.float32),
                   jax.ShapeDtypeStruct((1, VOCAB), jnp.float32)),
    )(codebook, dec_w1, dec_b1.reshape(1, D), dec_w2, dec_b2.reshape(1, D),
      dec_w3, dec_b3.reshape(1, PD))

    bspecs = [
        pl.BlockSpec((BLK, PD), lambda i: (i, 0)),
        full((PD, D)), full((1, D)),
        full((D, D)), full((1, D)),
        full((D, D)), full((1, D)),
        full((VOCAB, D)),
        full((1, VOCAB)),
        full((VOCAB, PD)),
    ]
    out_shapes = (
        jax.ShapeDtypeStruct((R, PD), jnp.float32),
        jax.ShapeDtypeStruct((G, 1, BLK), jnp.int32),
        jax.ShapeDtypeStruct((1, 2), jnp.float32),
    )
    out_specs = (
        pl.BlockSpec((BLK, PD), lambda i: (i, 0)),
        pl.BlockSpec((1, 1, BLK), lambda i: (i, 0, 0)),
        pl.BlockSpec((1, 2), lambda i: (0, 0)),
    )
    p_full, tok3, sums = pl.pallas_call(
        _main_body,
        grid=(G,),
        in_specs=bspecs,
        out_specs=out_specs,
        out_shape=out_shapes,
    )(praw, enc_w1, enc_b1.reshape(1, D), enc_w2, enc_b2.reshape(1, D),
      enc_w3, enc_b3.reshape(1, D), codebook, cn, ptable)

    tokens = tok3.reshape(B, N)

    lane = jnp.arange(RW, dtype=jnp.int32)
    wp_pat = (lane // LC).reshape(NV, 16)
    col_pat = (lane % LC).reshape(NV, 16)
    rec2d = _sc_unpatch(ptable, tok3.reshape(R), wp_pat, col_pat)
    recon = rec2d.reshape(B, H, W, C)
    recon_loss = sums[0, 0] / (B * H * W * C)
    vq_loss = sums[0, 1] / (R * D)
    return (recon, tokens, recon_loss, vq_loss, vq_loss)
